# Initial kernel scaffold; baseline (speedup 1.0000x reference)
import jax
import jax.numpy as jnp
from jax.experimental import pallas as pl


def kernel(vectors_source, vectors_target, src_idx, retrieval_window):
    raise NotImplementedError("write your pallas kernel here")


# speedup vs baseline: 12.8311x; 12.8311x over previous
import jax
import jax.numpy as jnp
from jax.experimental import pallas as pl
from jax.experimental.pallas import tpu as pltpu

_V1 = 4096
_V2 = 4096
_D = 128
_WLEN = 1000
_TILE = 128
_NTILES = 8
_NEG = -3.0e38
_BIGI = 1 << 30


def _bitonic_sort_rows(x):
    n = x.shape[-1]
    lane = jax.lax.broadcasted_iota(jnp.int32, x.shape, x.ndim - 1)
    k = 2
    while k <= n:
        j = k // 2
        while j >= 1:
            up = pltpu.roll(x, n - j, 1)
            dn = pltpu.roll(x, j, 1)
            is_lo = (lane & j) == 0
            partner = jnp.where(is_lo, up, dn)
            asc = (lane & k) == 0
            take_min = asc == is_lo
            x = jnp.where(take_min, jnp.minimum(x, partner),
                          jnp.maximum(x, partner))
            j //= 2
        k *= 2
    return x


def _body(vs_row_ref, vs_ref, vt_win_ref, vt_ref, win_ref, idx_ref,
          s_ref, bestv_ref, besti_ref):
    i = pl.program_id(0)

    @pl.when(i == 0)
    def _init():
        a = jax.lax.dot_general(
            vs_row_ref[...], vs_ref[...], (((1,), (1,)), ((), ())),
            preferred_element_type=jnp.float32)
        a = _bitonic_sort_rows(a)
        a = a / jnp.sqrt(jnp.sum(a * a))
        s_ref[...] = jnp.broadcast_to(a, (8, _V2)).astype(jnp.bfloat16)
        bestv_ref[0] = _NEG
        besti_ref[0] = _BIGI

    b = vt_win_ref[...]
    r = jax.lax.dot_general(
        b, vt_ref[...], (((1,), (1,)), ((), ())),
        preferred_element_type=jnp.float32)
    srt = _bitonic_sort_rows(r)
    srt = srt / jnp.sqrt(jnp.sum(srt * srt, axis=1, keepdims=True))
    vals8 = jax.lax.dot_general(
        srt.astype(jnp.bfloat16), s_ref[...], (((1,), (1,)), ((), ())),
        preferred_element_type=jnp.float32)
    vals = vals8[:, 0:1]
    win_ref[...] = vals

    gpos = jax.lax.broadcasted_iota(jnp.int32, (_TILE, 1), 0) + i * _TILE
    masked = jnp.where(gpos < _WLEN, vals, _NEG)
    lm = jnp.max(masked)
    la = jnp.min(jnp.where(masked == lm, gpos, _BIGI))
    better = lm > bestv_ref[0]
    besti_ref[0] = jnp.where(better, la, besti_ref[0])
    bestv_ref[0] = jnp.where(better, lm, bestv_ref[0])

    @pl.when(i == _NTILES - 1)
    def _fin():
        idx_ref[0] = besti_ref[0]


def kernel(vectors_source, vectors_target, src_idx, retrieval_window):
    src_idx = jnp.asarray(src_idx, jnp.int32)
    rw = jnp.asarray(retrieval_window, jnp.int32)
    start_u = jnp.maximum(0, src_idx - rw)
    start_c = jnp.clip(start_u, 0, _V2 - _WLEN)
    vs_row = jax.lax.dynamic_slice(
        vectors_source, (src_idx, jnp.int32(0)), (1, _D))
    vt_win = jax.lax.dynamic_slice(
        vectors_target, (start_c, jnp.int32(0)), (_NTILES * _TILE, _D))

    win_col, rel_idx = pl.pallas_call(
        _body,
        grid=(_NTILES,),
        in_specs=[
            pl.BlockSpec((1, _D), lambda i: (0, 0)),
            pl.BlockSpec((_V1, _D), lambda i: (0, 0)),
            pl.BlockSpec((_TILE, _D), lambda i: (i, 0)),
            pl.BlockSpec((_V2, _D), lambda i: (0, 0)),
        ],
        out_specs=[
            pl.BlockSpec((_TILE, 1), lambda i: (i, 0)),
            pl.BlockSpec(memory_space=pltpu.SMEM),
        ],
        out_shape=[
            jax.ShapeDtypeStruct((_NTILES * _TILE, 1), jnp.float32),
            jax.ShapeDtypeStruct((1,), jnp.int32),
        ],
        scratch_shapes=[
            pltpu.VMEM((8, _V2), jnp.bfloat16),
            pltpu.SMEM((1,), jnp.float32),
            pltpu.SMEM((1,), jnp.int32),
        ],
    )(vs_row, vectors_source, vt_win, vectors_target)

    window = win_col[:_WLEN, 0]
    target_idx = rel_idx[0] + start_u
    return (target_idx, window)


# speedup vs baseline: 19.7466x; 1.5390x over previous
import jax
import jax.numpy as jnp
from jax.experimental import pallas as pl
from jax.experimental.pallas import tpu as pltpu

_V1 = 4096
_V2 = 4096
_D = 128
_WLEN = 1000
_TILE = 128
_NTILES = 8
_NEG = -3.0e38
_BIGI = 1 << 30


def _bitonic_sort_rows(x):
    n = x.shape[-1]
    lane = jax.lax.broadcasted_iota(jnp.int32, (1, n), 1)
    k = 2
    while k <= n:
        j = k // 2
        while j >= 1:
            up = pltpu.roll(x, n - j, 1)
            dn = pltpu.roll(x, j, 1)
            is_lo = (lane & j) == 0
            partner = jnp.where(is_lo, up, dn)
            asc = (lane & k) == 0
            take_min = asc == is_lo
            x = jnp.where(take_min, jnp.minimum(x, partner),
                          jnp.maximum(x, partner))
            j //= 2
        k *= 2
    return x


def _body(vs_row_ref, vs_ref, vt_win_ref, vt_ref, win_ref, idx_ref,
          s_ref, bestv_ref, besti_ref):
    i = pl.program_id(0)

    @pl.when(i == 0)
    def _init():
        a = jax.lax.dot_general(
            vs_row_ref[...], vs_ref[...], (((1,), (1,)), ((), ())),
            preferred_element_type=jnp.float32)
        a = a / jnp.sqrt(jnp.sum(a * a))
        a = _bitonic_sort_rows(a.astype(jnp.bfloat16))
        s_ref[...] = jnp.broadcast_to(a, (8, _V2))
        bestv_ref[0] = _NEG
        besti_ref[0] = _BIGI

    b = vt_win_ref[...]
    r = jax.lax.dot_general(
        b, vt_ref[...], (((1,), (1,)), ((), ())),
        preferred_element_type=jnp.float32)
    rn = r / jnp.sqrt(jnp.sum(r * r, axis=1, keepdims=True))
    srt = _bitonic_sort_rows(rn.astype(jnp.bfloat16))
    vals8 = jax.lax.dot_general(
        srt, s_ref[...], (((1,), (1,)), ((), ())),
        preferred_element_type=jnp.float32)
    vals = vals8[:, 0:1]
    win_ref[...] = vals

    gpos = jax.lax.broadcasted_iota(jnp.int32, (_TILE, 1), 0) + i * _TILE
    masked = jnp.where(gpos < _WLEN, vals, _NEG)
    lm = jnp.max(masked)
    la = jnp.min(jnp.where(masked == lm, gpos, _BIGI))
    better = lm > bestv_ref[0]
    besti_ref[0] = jnp.where(better, la, besti_ref[0])
    bestv_ref[0] = jnp.where(better, lm, bestv_ref[0])

    @pl.when(i == _NTILES - 1)
    def _fin():
        idx_ref[0] = besti_ref[0]


def kernel(vectors_source, vectors_target, src_idx, retrieval_window):
    src_idx = jnp.asarray(src_idx, jnp.int32)
    rw = jnp.asarray(retrieval_window, jnp.int32)
    start_u = jnp.maximum(0, src_idx - rw)
    start_c = jnp.clip(start_u, 0, _V2 - _WLEN)
    vs_row = jax.lax.dynamic_slice(
        vectors_source, (src_idx, jnp.int32(0)), (1, _D))
    vt_win = jax.lax.dynamic_slice(
        vectors_target, (start_c, jnp.int32(0)), (_NTILES * _TILE, _D))

    win_col, rel_idx = pl.pallas_call(
        _body,
        grid=(_NTILES,),
        in_specs=[
            pl.BlockSpec((1, _D), lambda i: (0, 0)),
            pl.BlockSpec((_V1, _D), lambda i: (0, 0)),
            pl.BlockSpec((_TILE, _D), lambda i: (i, 0)),
            pl.BlockSpec((_V2, _D), lambda i: (0, 0)),
        ],
        out_specs=[
            pl.BlockSpec((_TILE, 1), lambda i: (i, 0)),
            pl.BlockSpec(memory_space=pltpu.SMEM),
        ],
        out_shape=[
            jax.ShapeDtypeStruct((_NTILES * _TILE, 1), jnp.float32),
            jax.ShapeDtypeStruct((1,), jnp.int32),
        ],
        scratch_shapes=[
            pltpu.VMEM((8, _V2), jnp.bfloat16),
            pltpu.SMEM((1,), jnp.float32),
            pltpu.SMEM((1,), jnp.int32),
        ],
    )(vs_row, vectors_source, vt_win, vectors_target)

    window = win_col[:_WLEN, 0]
    target_idx = rel_idx[0] + start_u
    return (target_idx, window)


# speedup vs baseline: 22.9050x; 1.1599x over previous
import jax
import jax.numpy as jnp
from jax.experimental import pallas as pl
from jax.experimental.pallas import tpu as pltpu

_V1 = 4096
_V2 = 4096
_D = 128
_WLEN = 1000
_TILE = 200
_NTILES = 5
_NEG = -3.0e38
_BIGI = 1 << 30


def _bitonic_sort_rows(x):
    n = x.shape[-1]
    lane = jax.lax.broadcasted_iota(jnp.int32, (1, n), 1)
    k = 2
    while k <= n:
        j = k // 2
        while j >= 1:
            up = pltpu.roll(x, n - j, 1)
            dn = pltpu.roll(x, j, 1)
            is_lo = (lane & j) == 0
            partner = jnp.where(is_lo, up, dn)
            asc = (lane & k) == 0
            take_min = asc == is_lo
            x = jnp.where(take_min, jnp.minimum(x, partner),
                          jnp.maximum(x, partner))
            j //= 2
        k *= 2
    return x


def _body(vs_row_ref, vs_ref, vt_win_ref, vt_ref, win_ref, idx_ref,
          s_ref, bestv_ref, besti_ref):
    i = pl.program_id(0)

    @pl.when(i == 0)
    def _init():
        a = jax.lax.dot_general(
            vs_row_ref[...], vs_ref[...], (((1,), (1,)), ((), ())),
            preferred_element_type=jnp.float32)
        a = a / jnp.sqrt(jnp.sum(a * a))
        a = _bitonic_sort_rows(a.astype(jnp.bfloat16))
        s_ref[...] = jnp.broadcast_to(a, (8, _V2))
        bestv_ref[0] = _NEG
        besti_ref[0] = _BIGI

    b = vt_win_ref[...]
    r = jax.lax.dot_general(
        b, vt_ref[...], (((1,), (1,)), ((), ())),
        preferred_element_type=jnp.float32)
    rn = r / jnp.sqrt(jnp.sum(r * r, axis=1, keepdims=True))
    srt = _bitonic_sort_rows(rn.astype(jnp.bfloat16))
    vals8 = jax.lax.dot_general(
        srt, s_ref[...], (((1,), (1,)), ((), ())),
        preferred_element_type=jnp.float32)
    vals = vals8[:, 0:1]
    win_ref[...] = vals

    gpos = jax.lax.broadcasted_iota(jnp.int32, (_TILE, 1), 0) + i * _TILE
    masked = jnp.where(gpos < _WLEN, vals, _NEG)
    lm = jnp.max(masked)
    la = jnp.min(jnp.where(masked == lm, gpos, _BIGI))
    better = lm > bestv_ref[0]
    besti_ref[0] = jnp.where(better, la, besti_ref[0])
    bestv_ref[0] = jnp.where(better, lm, bestv_ref[0])

    @pl.when(i == _NTILES - 1)
    def _fin():
        idx_ref[0] = besti_ref[0]


def kernel(vectors_source, vectors_target, src_idx, retrieval_window):
    src_idx = jnp.asarray(src_idx, jnp.int32)
    rw = jnp.asarray(retrieval_window, jnp.int32)
    start_u = jnp.maximum(0, src_idx - rw)
    start_c = jnp.clip(start_u, 0, _V2 - _WLEN)
    vs_row = jax.lax.dynamic_slice(
        vectors_source, (src_idx, jnp.int32(0)), (1, _D))
    vt_win = jax.lax.dynamic_slice(
        vectors_target, (start_c, jnp.int32(0)), (_NTILES * _TILE, _D))

    win_col, rel_idx = pl.pallas_call(
        _body,
        grid=(_NTILES,),
        in_specs=[
            pl.BlockSpec((1, _D), lambda i: (0, 0)),
            pl.BlockSpec((_V1, _D), lambda i: (0, 0)),
            pl.BlockSpec((_TILE, _D), lambda i: (i, 0)),
            pl.BlockSpec((_V2, _D), lambda i: (0, 0)),
        ],
        out_specs=[
            pl.BlockSpec((_TILE, 1), lambda i: (i, 0)),
            pl.BlockSpec(memory_space=pltpu.SMEM),
        ],
        out_shape=[
            jax.ShapeDtypeStruct((_NTILES * _TILE, 1), jnp.float32),
            jax.ShapeDtypeStruct((1,), jnp.int32),
        ],
        scratch_shapes=[
            pltpu.VMEM((8, _V2), jnp.bfloat16),
            pltpu.SMEM((1,), jnp.float32),
            pltpu.SMEM((1,), jnp.int32),
        ],
    )(vs_row, vectors_source, vt_win, vectors_target)

    window = win_col[:_WLEN, 0]
    target_idx = rel_idx[0] + start_u
    return (target_idx, window)
